# baseline (speedup 1.0000x reference)
import functools

import jax
import jax.numpy as jnp
from jax import lax
from jax.experimental import pallas as pl
from jax.experimental.pallas import tpu as pltpu
from jax.experimental.pallas import tpu_sc as plsc

VOCAB = 100000
VEC_DIM = 128
HIDDEN = 256
LABELS = 16
BATCH = 4096
SEQ = 200

NC = 2
NS = 16
NW = NC * NS
B_PER_W = BATCH // NW
HALF = SEQ // 2
ROWS_PER_W = B_PER_W * 2


PACK_BLK = 2000


def _pack_kernel(e_ref, out_ref):
  bf = e_ref[...].astype(jnp.bfloat16)
  bits = lax.bitcast_convert_type(bf, jnp.uint16).astype(jnp.int32)
  lo = bits[:, :64]
  hi = bits[:, 64:]
  out_ref[...] = lo | (hi << 16)


def _pack_table(embed):
  return pl.pallas_call(
      _pack_kernel,
      grid=(VOCAB // PACK_BLK,),
      in_specs=[pl.BlockSpec((PACK_BLK, VEC_DIM), lambda i: (i, 0))],
      out_specs=pl.BlockSpec((PACK_BLK, VEC_DIM // 2), lambda i: (i, 0)),
      out_shape=jax.ShapeDtypeStruct((VOCAB, VEC_DIM // 2), jnp.int32),
  )(embed)


def _sc_gather_sum(embed, xr):
  mesh = plsc.VectorSubcoreMesh(core_axis_name="c", subcore_axis_name="s")

  @functools.partial(
      pl.kernel,
      mesh=mesh,
      out_type=jax.ShapeDtypeStruct((BATCH, VEC_DIM), jnp.float32),
      compiler_params=pltpu.CompilerParams(use_tc_tiling_on_sc=False),
      scratch_types=[
          pltpu.VMEM((ROWS_PER_W, HALF), jnp.int32),
          pltpu.VMEM((HALF, VEC_DIM // 2), jnp.int32),
          pltpu.VMEM((HALF, VEC_DIM // 2), jnp.int32),
          pltpu.VMEM((HALF, VEC_DIM // 2), jnp.int32),
          pltpu.VMEM((HALF, VEC_DIM // 2), jnp.int32),
          pltpu.VMEM((HALF, VEC_DIM // 2), jnp.int32),
          pltpu.VMEM((HALF, VEC_DIM // 2), jnp.int32),
          pltpu.VMEM((B_PER_W, VEC_DIM), jnp.float32),
          pltpu.SemaphoreType.DMA,
          pltpu.SemaphoreType.DMA,
          pltpu.SemaphoreType.DMA,
          pltpu.SemaphoreType.DMA,
          pltpu.SemaphoreType.DMA,
          pltpu.SemaphoreType.DMA,
      ],
  )
  def k(embed_hbm, xr_hbm, out_hbm, idx_v, b0, b1, b2, b3, b4, b5, m_local,
        s0, s1, s2, s3, s4, s5):
    bufs = (b0, b1, b2, b3, b4, b5)
    sems = (s0, s1, s2, s3, s4, s5)
    wid = lax.axis_index("s") * NC + lax.axis_index("c")
    pltpu.sync_copy(xr_hbm.at[pl.ds(wid * ROWS_PER_W, ROWS_PER_W)], idx_v)

    def gather_half(r, slot):
      pltpu.make_async_copy(
          embed_hbm.at[idx_v.at[r]], bufs[slot], sems[slot]).start()

    def wait_half(slot):
      pltpu.make_async_copy(
          embed_hbm.at[idx_v.at[0]], bufs[slot], sems[slot]).wait()

    def reduce_half(buf, accs):
      def body(j4, accs):
        for u in range(4):
          j = 4 * j4 + u
          accs = list(accs)
          for d in range(4):
            w = buf[j, d * 16:(d + 1) * 16]
            a = lax.bitcast_convert_type(lax.shift_left(w, 16), jnp.float32)
            b = lax.bitcast_convert_type(
                lax.bitwise_and(w, jnp.int32(-65536)), jnp.float32)
            accs[d] = accs[d] + a
            accs[d + 4] = accs[d + 4] + b
          accs = tuple(accs)
        return accs
      return lax.fori_loop(0, HALF // 4, body, accs)

    for slot in range(5):
      gather_half(slot, slot)

    def outer(g, carry):
      r0 = 6 * g
      accs = None
      for s in range(6):
        @pl.when(r0 + s + 5 < ROWS_PER_W)
        def _(s=s):
          gather_half(r0 + s + 5, (s + 5) % 6)

        wait_half(s)
        if s % 2 == 0:
          accs = tuple(jnp.zeros((16,), jnp.float32) for _ in range(8))
        accs = reduce_half(bufs[s], accs)
        if s % 2 == 1:
          b = 3 * g + s // 2
          for d in range(8):
            m_local[b, d * 16:(d + 1) * 16] = accs[d]
      return carry

    n_full = ROWS_PER_W // 6
    lax.fori_loop(0, n_full, outer, 0)
    accs = None
    for s in range(ROWS_PER_W % 6):
      wait_half(s)
      if s % 2 == 0:
        accs = tuple(jnp.zeros((16,), jnp.float32) for _ in range(8))
      accs = reduce_half(bufs[s], accs)
      if s % 2 == 1:
        b = 3 * n_full + s // 2
        for d in range(8):
          m_local[b, d * 16:(d + 1) * 16] = accs[d]
    pltpu.sync_copy(m_local, out_hbm.at[pl.ds(wid * B_PER_W, B_PER_W)])

  return k(embed, xr)


def _mlp_kernel(msum_ref, w1_ref, b1_ref, g_ref, be_ref, w2_ref, b2_ref,
                out_ref):
  m = msum_ref[...] * (1.0 / SEQ)
  h = lax.dot_general(m, w1_ref[...], (((1,), (1,)), ((), ())),
                      preferred_element_type=jnp.float32)
  h = h + b1_ref[...]
  mu = jnp.mean(h, axis=0, keepdims=True)
  d = h - mu
  var = jnp.mean(d * d, axis=0, keepdims=True)
  hn = d * lax.rsqrt(var + 1e-5) * g_ref[...] + be_ref[...]
  a = jnp.maximum(hn, 0.0)
  out_ref[...] = lax.dot_general(a, w2_ref[...], (((1,), (1,)), ((), ())),
                                 preferred_element_type=jnp.float32) + b2_ref[...]


def kernel(X, embed, W1, b1, gamma, beta, W2, b2):
  xr = X.astype(jnp.int32).reshape(BATCH * 2, HALF)
  packed = _pack_table(embed)
  msum = _sc_gather_sum(packed, xr)
  out = pl.pallas_call(
      _mlp_kernel,
      out_shape=jax.ShapeDtypeStruct((BATCH, LABELS), jnp.float32),
  )(msum, W1, b1.reshape(1, HIDDEN), gamma.reshape(1, HIDDEN),
    beta.reshape(1, HIDDEN), W2, b2.reshape(1, LABELS))
  return out


# speedup vs baseline: 1.1912x; 1.1912x over previous
import functools

import jax
import jax.numpy as jnp
from jax import lax
from jax.experimental import pallas as pl
from jax.experimental.pallas import tpu as pltpu
from jax.experimental.pallas import tpu_sc as plsc

VOCAB = 100000
VEC_DIM = 128
HIDDEN = 256
LABELS = 16
BATCH = 4096
SEQ = 200

NC = 2
NS = 16
NW = NC * NS
B_PER_W = BATCH // NW
HALF = SEQ // 2
ROWS_PER_W = B_PER_W * 2


PACK_BLK = 2000


def _pack_kernel(e_ref, out_ref):
  bf = e_ref[...].astype(jnp.bfloat16)
  bits = lax.bitcast_convert_type(bf, jnp.uint16).astype(jnp.int32)
  lo = bits[:, :64]
  hi = bits[:, 64:]
  packed = lo | (hi << 16)
  out_ref[...] = jnp.concatenate(
      [packed, jnp.zeros_like(packed)], axis=1)


def _pack_table(embed):
  return pl.pallas_call(
      _pack_kernel,
      grid=(VOCAB // PACK_BLK,),
      in_specs=[pl.BlockSpec((PACK_BLK, VEC_DIM), lambda i: (i, 0))],
      out_specs=pl.BlockSpec((PACK_BLK, VEC_DIM), lambda i: (i, 0)),
      out_shape=jax.ShapeDtypeStruct((VOCAB, VEC_DIM), jnp.int32),
  )(embed)


def _sc_gather_sum(embed, xr):
  mesh = plsc.VectorSubcoreMesh(core_axis_name="c", subcore_axis_name="s")

  @functools.partial(
      pl.kernel,
      mesh=mesh,
      out_type=jax.ShapeDtypeStruct((BATCH, VEC_DIM), jnp.float32),
      compiler_params=pltpu.CompilerParams(use_tc_tiling_on_sc=False),
      scratch_types=[
          pltpu.VMEM((ROWS_PER_W, HALF), jnp.int32),
          pltpu.VMEM((HALF, VEC_DIM // 2), jnp.int32),
          pltpu.VMEM((HALF, VEC_DIM // 2), jnp.int32),
          pltpu.VMEM((HALF, VEC_DIM // 2), jnp.int32),
          pltpu.VMEM((HALF, VEC_DIM // 2), jnp.int32),
          pltpu.VMEM((HALF, VEC_DIM // 2), jnp.int32),
          pltpu.VMEM((HALF, VEC_DIM // 2), jnp.int32),
          pltpu.VMEM((B_PER_W, VEC_DIM), jnp.float32),
          pltpu.SemaphoreType.DMA,
          pltpu.SemaphoreType.DMA,
          pltpu.SemaphoreType.DMA,
          pltpu.SemaphoreType.DMA,
          pltpu.SemaphoreType.DMA,
          pltpu.SemaphoreType.DMA,
      ],
  )
  def k(embed_hbm, xr_hbm, out_hbm, idx_v, b0, b1, b2, b3, b4, b5, m_local,
        s0, s1, s2, s3, s4, s5):
    bufs = (b0, b1, b2, b3, b4, b5)
    sems = (s0, s1, s2, s3, s4, s5)
    wid = lax.axis_index("s") * NC + lax.axis_index("c")
    pltpu.sync_copy(xr_hbm.at[pl.ds(wid * ROWS_PER_W, ROWS_PER_W)], idx_v)

    def gather_half(r, slot):
      pltpu.make_async_copy(
          embed_hbm.at[idx_v.at[r]], bufs[slot], sems[slot]).start()

    def wait_half(slot):
      pltpu.make_async_copy(
          embed_hbm.at[idx_v.at[0]], bufs[slot], sems[slot]).wait()

    def reduce_half(buf, accs):
      def body(j4, accs):
        for u in range(4):
          j = 4 * j4 + u
          accs = list(accs)
          for d in range(4):
            w = buf[j, d * 16:(d + 1) * 16]
            a = lax.bitcast_convert_type(lax.shift_left(w, 16), jnp.float32)
            b = lax.bitcast_convert_type(
                lax.bitwise_and(w, jnp.int32(-65536)), jnp.float32)
            accs[d] = accs[d] + a
            accs[d + 4] = accs[d + 4] + b
          accs = tuple(accs)
        return accs
      return lax.fori_loop(0, HALF // 4, body, accs)

    for slot in range(5):
      gather_half(slot, slot)

    def outer(g, carry):
      r0 = 6 * g
      accs = None
      for s in range(6):
        @pl.when(r0 + s + 5 < ROWS_PER_W)
        def _(s=s):
          gather_half(r0 + s + 5, (s + 5) % 6)

        wait_half(s)
        if s % 2 == 0:
          accs = tuple(jnp.zeros((16,), jnp.float32) for _ in range(8))
        accs = reduce_half(bufs[s], accs)
        if s % 2 == 1:
          b = 3 * g + s // 2
          for d in range(8):
            m_local[b, d * 16:(d + 1) * 16] = accs[d]
      return carry

    n_full = ROWS_PER_W // 6
    lax.fori_loop(0, n_full, outer, 0)
    accs = None
    for s in range(ROWS_PER_W % 6):
      wait_half(s)
      if s % 2 == 0:
        accs = tuple(jnp.zeros((16,), jnp.float32) for _ in range(8))
      accs = reduce_half(bufs[s], accs)
      if s % 2 == 1:
        b = 3 * n_full + s // 2
        for d in range(8):
          m_local[b, d * 16:(d + 1) * 16] = accs[d]
    pltpu.sync_copy(m_local, out_hbm.at[pl.ds(wid * B_PER_W, B_PER_W)])

  return k(embed, xr)


def _mlp_kernel(msum_ref, w1_ref, b1_ref, g_ref, be_ref, w2_ref, b2_ref,
                out_ref):
  m = msum_ref[...] * (1.0 / SEQ)
  h = lax.dot_general(m, w1_ref[...], (((1,), (1,)), ((), ())),
                      preferred_element_type=jnp.float32)
  h = h + b1_ref[...]
  mu = jnp.mean(h, axis=0, keepdims=True)
  d = h - mu
  var = jnp.mean(d * d, axis=0, keepdims=True)
  hn = d * lax.rsqrt(var + 1e-5) * g_ref[...] + be_ref[...]
  a = jnp.maximum(hn, 0.0)
  out_ref[...] = lax.dot_general(a, w2_ref[...], (((1,), (1,)), ((), ())),
                                 preferred_element_type=jnp.float32) + b2_ref[...]


def kernel(X, embed, W1, b1, gamma, beta, W2, b2):
  xr = (X.astype(jnp.int32) * 2).reshape(BATCH * 2, HALF)
  packed = _pack_table(embed).reshape(2 * VOCAB, VEC_DIM // 2)
  msum = _sc_gather_sum(packed, xr)
  out = pl.pallas_call(
      _mlp_kernel,
      out_shape=jax.ShapeDtypeStruct((BATCH, LABELS), jnp.float32),
  )(msum, W1, b1.reshape(1, HIDDEN), gamma.reshape(1, HIDDEN),
    beta.reshape(1, HIDDEN), W2, b2.reshape(1, LABELS))
  return out


# speedup vs baseline: 1.2240x; 1.0275x over previous
import functools

import jax
import jax.numpy as jnp
from jax import lax
from jax.experimental import pallas as pl
from jax.experimental.pallas import tpu as pltpu
from jax.experimental.pallas import tpu_sc as plsc

VOCAB = 100000
VEC_DIM = 128
HIDDEN = 256
LABELS = 16
BATCH = 4096
SEQ = 200

NC = 2
NS = 16
NW = NC * NS
B_PER_W = BATCH // NW
HALF = SEQ // 2
ROWS_PER_W = B_PER_W * 2


PACK_BLK = 1000


def _pack_words(e):
  bf = e.astype(jnp.bfloat16)
  bits = lax.bitcast_convert_type(bf, jnp.uint16).astype(jnp.int32)
  return bits[:, :64] | (bits[:, 64:] << 16)


def _pack_kernel(e1_ref, e2_ref, out_ref):
  out_ref[...] = jnp.concatenate(
      [_pack_words(e1_ref[...]), _pack_words(e2_ref[...])], axis=1)


def _pack_table(embed):
  hblk = (VOCAB // 2) // PACK_BLK
  return pl.pallas_call(
      _pack_kernel,
      grid=(hblk,),
      in_specs=[
          pl.BlockSpec((PACK_BLK, VEC_DIM), lambda i: (i, 0)),
          pl.BlockSpec((PACK_BLK, VEC_DIM), lambda i, h=hblk: (i + h, 0)),
      ],
      out_specs=pl.BlockSpec((PACK_BLK, VEC_DIM), lambda i: (i, 0)),
      out_shape=jax.ShapeDtypeStruct((VOCAB // 2, VEC_DIM), jnp.int32),
  )(embed, embed)


def _sc_gather_sum(embed, xr):
  mesh = plsc.VectorSubcoreMesh(core_axis_name="c", subcore_axis_name="s")

  @functools.partial(
      pl.kernel,
      mesh=mesh,
      out_type=jax.ShapeDtypeStruct((BATCH, VEC_DIM), jnp.float32),
      compiler_params=pltpu.CompilerParams(use_tc_tiling_on_sc=False),
      scratch_types=[
          pltpu.VMEM((ROWS_PER_W, HALF), jnp.int32),
          pltpu.VMEM((HALF, VEC_DIM // 2), jnp.int32),
          pltpu.VMEM((HALF, VEC_DIM // 2), jnp.int32),
          pltpu.VMEM((HALF, VEC_DIM // 2), jnp.int32),
          pltpu.VMEM((HALF, VEC_DIM // 2), jnp.int32),
          pltpu.VMEM((HALF, VEC_DIM // 2), jnp.int32),
          pltpu.VMEM((HALF, VEC_DIM // 2), jnp.int32),
          pltpu.VMEM((B_PER_W, VEC_DIM), jnp.float32),
          pltpu.SemaphoreType.DMA,
          pltpu.SemaphoreType.DMA,
          pltpu.SemaphoreType.DMA,
          pltpu.SemaphoreType.DMA,
          pltpu.SemaphoreType.DMA,
          pltpu.SemaphoreType.DMA,
      ],
  )
  def k(embed_hbm, xr_hbm, out_hbm, idx_v, b0, b1, b2, b3, b4, b5, m_local,
        s0, s1, s2, s3, s4, s5):
    bufs = (b0, b1, b2, b3, b4, b5)
    sems = (s0, s1, s2, s3, s4, s5)
    wid = lax.axis_index("s") * NC + lax.axis_index("c")
    pltpu.sync_copy(xr_hbm.at[pl.ds(wid * ROWS_PER_W, ROWS_PER_W)], idx_v)

    def gather_half(r, slot):
      pltpu.make_async_copy(
          embed_hbm.at[idx_v.at[r]], bufs[slot], sems[slot]).start()

    def wait_half(slot):
      pltpu.make_async_copy(
          embed_hbm.at[idx_v.at[0]], bufs[slot], sems[slot]).wait()

    def reduce_half(buf, accs):
      def body(j4, accs):
        for u in range(4):
          j = 4 * j4 + u
          accs = list(accs)
          for d in range(4):
            w = buf[j, d * 16:(d + 1) * 16]
            a = lax.bitcast_convert_type(lax.shift_left(w, 16), jnp.float32)
            b = lax.bitcast_convert_type(
                lax.bitwise_and(w, jnp.int32(-65536)), jnp.float32)
            accs[d] = accs[d] + a
            accs[d + 4] = accs[d + 4] + b
          accs = tuple(accs)
        return accs
      return lax.fori_loop(0, HALF // 4, body, accs)

    for slot in range(5):
      gather_half(slot, slot)

    def outer(g, carry):
      r0 = 6 * g
      accs = None
      for s in range(6):
        @pl.when(r0 + s + 5 < ROWS_PER_W)
        def _(s=s):
          gather_half(r0 + s + 5, (s + 5) % 6)

        wait_half(s)
        if s % 2 == 0:
          accs = tuple(jnp.zeros((16,), jnp.float32) for _ in range(8))
        accs = reduce_half(bufs[s], accs)
        if s % 2 == 1:
          b = 3 * g + s // 2
          for d in range(8):
            m_local[b, d * 16:(d + 1) * 16] = accs[d]
      return carry

    n_full = ROWS_PER_W // 6
    lax.fori_loop(0, n_full, outer, 0)
    accs = None
    for s in range(ROWS_PER_W % 6):
      wait_half(s)
      if s % 2 == 0:
        accs = tuple(jnp.zeros((16,), jnp.float32) for _ in range(8))
      accs = reduce_half(bufs[s], accs)
      if s % 2 == 1:
        b = 3 * n_full + s // 2
        for d in range(8):
          m_local[b, d * 16:(d + 1) * 16] = accs[d]
    pltpu.sync_copy(m_local, out_hbm.at[pl.ds(wid * B_PER_W, B_PER_W)])

  return k(embed, xr)


def _mlp_kernel(msum_ref, w1_ref, b1_ref, g_ref, be_ref, w2_ref, b2_ref,
                out_ref):
  m = msum_ref[...] * (1.0 / SEQ)
  h = lax.dot_general(m, w1_ref[...], (((1,), (1,)), ((), ())),
                      preferred_element_type=jnp.float32)
  h = h + b1_ref[...]
  mu = jnp.mean(h, axis=0, keepdims=True)
  d = h - mu
  var = jnp.mean(d * d, axis=0, keepdims=True)
  hn = d * lax.rsqrt(var + 1e-5) * g_ref[...] + be_ref[...]
  a = jnp.maximum(hn, 0.0)
  out_ref[...] = lax.dot_general(a, w2_ref[...], (((1,), (1,)), ((), ())),
                                 preferred_element_type=jnp.float32) + b2_ref[...]


def kernel(X, embed, W1, b1, gamma, beta, W2, b2):
  xi = X.astype(jnp.int32)
  xr = jnp.where(xi < VOCAB // 2, 2 * xi,
                 2 * xi - (VOCAB - 1)).reshape(BATCH * 2, HALF)
  packed = _pack_table(embed).reshape(VOCAB, VEC_DIM // 2)
  msum = _sc_gather_sum(packed, xr)
  out = pl.pallas_call(
      _mlp_kernel,
      out_shape=jax.ShapeDtypeStruct((BATCH, LABELS), jnp.float32),
  )(msum, W1, b1.reshape(1, HIDDEN), gamma.reshape(1, HIDDEN),
    beta.reshape(1, HIDDEN), W2, b2.reshape(1, LABELS))
  return out


# speedup vs baseline: 1.3129x; 1.0727x over previous
import functools

import jax
import jax.numpy as jnp
from jax import lax
from jax.experimental import pallas as pl
from jax.experimental.pallas import tpu as pltpu
from jax.experimental.pallas import tpu_sc as plsc

VOCAB = 100000
VEC_DIM = 128
HIDDEN = 256
LABELS = 16
BATCH = 4096
SEQ = 200

NC = 2
NS = 16
NW = NC * NS
B_PER_W = BATCH // NW
HALF = SEQ // 2
ROWS_PER_W = B_PER_W * 2


PACK_BLK = 2000


def _pack_words(e):
  bf = e.astype(jnp.bfloat16)
  bits = lax.bitcast_convert_type(bf, jnp.uint16).astype(jnp.int32)
  return bits[:, :64] | (bits[:, 64:] << 16)


def _pack_kernel(e1_ref, e2_ref, out_ref):
  out_ref[...] = jnp.concatenate(
      [_pack_words(e1_ref[...]), _pack_words(e2_ref[...])], axis=1)


def _pack_table(embed):
  hblk = (VOCAB // 2) // PACK_BLK
  return pl.pallas_call(
      _pack_kernel,
      grid=(hblk,),
      in_specs=[
          pl.BlockSpec((PACK_BLK, VEC_DIM), lambda i: (i, 0)),
          pl.BlockSpec((PACK_BLK, VEC_DIM), lambda i, h=hblk: (i + h, 0)),
      ],
      out_specs=pl.BlockSpec((PACK_BLK, VEC_DIM), lambda i: (i, 0)),
      out_shape=jax.ShapeDtypeStruct((VOCAB // 2, VEC_DIM), jnp.int32),
  )(embed, embed)


def _sc_gather_sum(embed, xr):
  mesh = plsc.VectorSubcoreMesh(core_axis_name="c", subcore_axis_name="s")

  @functools.partial(
      pl.kernel,
      mesh=mesh,
      out_type=jax.ShapeDtypeStruct((BATCH, VEC_DIM), jnp.float32),
      compiler_params=pltpu.CompilerParams(use_tc_tiling_on_sc=False),
      scratch_types=[
          pltpu.VMEM((ROWS_PER_W, HALF), jnp.int32),
          pltpu.VMEM((HALF, VEC_DIM // 2), jnp.int32),
          pltpu.VMEM((HALF, VEC_DIM // 2), jnp.int32),
          pltpu.VMEM((HALF, VEC_DIM // 2), jnp.int32),
          pltpu.VMEM((HALF, VEC_DIM // 2), jnp.int32),
          pltpu.VMEM((HALF, VEC_DIM // 2), jnp.int32),
          pltpu.VMEM((HALF, VEC_DIM // 2), jnp.int32),
          pltpu.VMEM((B_PER_W, VEC_DIM), jnp.float32),
          pltpu.SemaphoreType.DMA,
          pltpu.SemaphoreType.DMA,
          pltpu.SemaphoreType.DMA,
          pltpu.SemaphoreType.DMA,
          pltpu.SemaphoreType.DMA,
          pltpu.SemaphoreType.DMA,
      ],
  )
  def k(embed_hbm, xr_hbm, out_hbm, idx_v, b0, b1, b2, b3, b4, b5, m_local,
        s0, s1, s2, s3, s4, s5):
    bufs = (b0, b1, b2, b3, b4, b5)
    sems = (s0, s1, s2, s3, s4, s5)
    wid = lax.axis_index("s") * NC + lax.axis_index("c")
    pltpu.sync_copy(xr_hbm.at[pl.ds(wid * ROWS_PER_W, ROWS_PER_W)], idx_v)

    def gather_half(r, slot):
      pltpu.make_async_copy(
          embed_hbm.at[idx_v.at[r]], bufs[slot], sems[slot]).start()

    def wait_half(slot):
      pltpu.make_async_copy(
          embed_hbm.at[idx_v.at[0]], bufs[slot], sems[slot]).wait()

    def reduce_half(buf, accs):
      def body(j4, accs):
        for u in range(4):
          j = 4 * j4 + u
          accs = list(accs)
          for d in range(4):
            w = buf[j, d * 16:(d + 1) * 16]
            a = lax.bitcast_convert_type(lax.shift_left(w, 16), jnp.float32)
            b = lax.bitcast_convert_type(
                lax.bitwise_and(w, jnp.int32(-65536)), jnp.float32)
            accs[d] = accs[d] + a
            accs[d + 4] = accs[d + 4] + b
          accs = tuple(accs)
        return accs
      return lax.fori_loop(0, HALF // 4, body, accs)

    for slot in range(5):
      gather_half(slot, slot)

    def outer(g, carry):
      r0 = 6 * g
      accs = None
      for s in range(6):
        @pl.when(r0 + s + 5 < ROWS_PER_W)
        def _(s=s):
          gather_half(r0 + s + 5, (s + 5) % 6)

        wait_half(s)
        if s % 2 == 0:
          accs = tuple(jnp.zeros((16,), jnp.float32) for _ in range(8))
        accs = reduce_half(bufs[s], accs)
        if s % 2 == 1:
          b = 3 * g + s // 2
          for d in range(8):
            m_local[b, d * 16:(d + 1) * 16] = accs[d]
      return carry

    n_full = ROWS_PER_W // 6
    lax.fori_loop(0, n_full, outer, 0)
    accs = None
    for s in range(ROWS_PER_W % 6):
      wait_half(s)
      if s % 2 == 0:
        accs = tuple(jnp.zeros((16,), jnp.float32) for _ in range(8))
      accs = reduce_half(bufs[s], accs)
      if s % 2 == 1:
        b = 3 * n_full + s // 2
        for d in range(8):
          m_local[b, d * 16:(d + 1) * 16] = accs[d]
    pltpu.sync_copy(m_local, out_hbm.at[pl.ds(wid * B_PER_W, B_PER_W)])

  return k(embed, xr)


def _mlp_kernel(msum_ref, w1_ref, b1_ref, g_ref, be_ref, w2_ref, b2_ref,
                out_ref):
  m = msum_ref[...] * (1.0 / SEQ)
  h = lax.dot_general(m, w1_ref[...], (((1,), (1,)), ((), ())),
                      preferred_element_type=jnp.float32)
  h = h + b1_ref[...]
  mu = jnp.mean(h, axis=0, keepdims=True)
  d = h - mu
  var = jnp.mean(d * d, axis=0, keepdims=True)
  hn = d * lax.rsqrt(var + 1e-5) * g_ref[...] + be_ref[...]
  a = jnp.maximum(hn, 0.0)
  out_ref[...] = lax.dot_general(a, w2_ref[...], (((1,), (1,)), ((), ())),
                                 preferred_element_type=jnp.float32) + b2_ref[...]


def kernel(X, embed, W1, b1, gamma, beta, W2, b2):
  xi = X.astype(jnp.int32)
  xr = jnp.where(xi < VOCAB // 2, 2 * xi,
                 2 * xi - (VOCAB - 1)).reshape(BATCH * 2, HALF)
  packed = _pack_table(embed).reshape(VOCAB, VEC_DIM // 2)
  msum = _sc_gather_sum(packed, xr)
  out = pl.pallas_call(
      _mlp_kernel,
      out_shape=jax.ShapeDtypeStruct((BATCH, LABELS), jnp.float32),
  )(msum, W1, b1.reshape(1, HIDDEN), gamma.reshape(1, HIDDEN),
    beta.reshape(1, HIDDEN), W2, b2.reshape(1, LABELS))
  return out


# speedup vs baseline: 1.3719x; 1.0449x over previous
import functools

import jax
import jax.numpy as jnp
from jax import lax
from jax.experimental import pallas as pl
from jax.experimental.pallas import tpu as pltpu
from jax.experimental.pallas import tpu_sc as plsc

VOCAB = 100000
VEC_DIM = 128
HIDDEN = 256
LABELS = 16
BATCH = 4096
SEQ = 200

NC = 2
NS = 16
NW = NC * NS
B_PER_W = BATCH // NW
HALF = SEQ // 2
ROWS_PER_W = B_PER_W * 2


PACK_BLK = 5000


def _pack_words(e):
  bf = e.astype(jnp.bfloat16)
  bits = lax.bitcast_convert_type(bf, jnp.uint16).astype(jnp.int32)
  return bits[:, :64] | (bits[:, 64:] << 16)


def _pack_kernel(e1_ref, e2_ref, out_ref):
  out_ref[...] = jnp.concatenate(
      [_pack_words(e1_ref[...]), _pack_words(e2_ref[...])], axis=1)


def _pack_table(embed):
  hblk = (VOCAB // 2) // PACK_BLK
  return pl.pallas_call(
      _pack_kernel,
      grid=(hblk,),
      in_specs=[
          pl.BlockSpec((PACK_BLK, VEC_DIM), lambda i: (i, 0)),
          pl.BlockSpec((PACK_BLK, VEC_DIM), lambda i, h=hblk: (i + h, 0)),
      ],
      out_specs=pl.BlockSpec((PACK_BLK, VEC_DIM), lambda i: (i, 0)),
      out_shape=jax.ShapeDtypeStruct((VOCAB // 2, VEC_DIM), jnp.int32),
  )(embed, embed)


def _sc_gather_sum(embed, xr):
  mesh = plsc.VectorSubcoreMesh(core_axis_name="c", subcore_axis_name="s")

  @functools.partial(
      pl.kernel,
      mesh=mesh,
      out_type=jax.ShapeDtypeStruct((BATCH, VEC_DIM), jnp.float32),
      compiler_params=pltpu.CompilerParams(use_tc_tiling_on_sc=False),
      scratch_types=[
          pltpu.VMEM((ROWS_PER_W, HALF), jnp.int32),
          pltpu.VMEM((HALF, VEC_DIM // 2), jnp.int32),
          pltpu.VMEM((HALF, VEC_DIM // 2), jnp.int32),
          pltpu.VMEM((HALF, VEC_DIM // 2), jnp.int32),
          pltpu.VMEM((HALF, VEC_DIM // 2), jnp.int32),
          pltpu.VMEM((HALF, VEC_DIM // 2), jnp.int32),
          pltpu.VMEM((HALF, VEC_DIM // 2), jnp.int32),
          pltpu.VMEM((B_PER_W, VEC_DIM), jnp.float32),
          pltpu.SemaphoreType.DMA,
          pltpu.SemaphoreType.DMA,
          pltpu.SemaphoreType.DMA,
          pltpu.SemaphoreType.DMA,
          pltpu.SemaphoreType.DMA,
          pltpu.SemaphoreType.DMA,
      ],
  )
  def k(embed_hbm, xr_hbm, out_hbm, idx_v, b0, b1, b2, b3, b4, b5, m_local,
        s0, s1, s2, s3, s4, s5):
    bufs = (b0, b1, b2, b3, b4, b5)
    sems = (s0, s1, s2, s3, s4, s5)
    wid = lax.axis_index("s") * NC + lax.axis_index("c")
    pltpu.sync_copy(xr_hbm.at[pl.ds(wid * ROWS_PER_W, ROWS_PER_W)], idx_v)

    def gather_half(r, slot):
      pltpu.make_async_copy(
          embed_hbm.at[idx_v.at[r]], bufs[slot], sems[slot]).start()

    def wait_half(slot):
      pltpu.make_async_copy(
          embed_hbm.at[idx_v.at[0]], bufs[slot], sems[slot]).wait()

    def reduce_half(buf, accs):
      def body(j4, accs):
        for u in range(4):
          j = 4 * j4 + u
          accs = list(accs)
          for d in range(4):
            w = buf[j, d * 16:(d + 1) * 16]
            a = lax.bitcast_convert_type(lax.shift_left(w, 16), jnp.float32)
            b = lax.bitcast_convert_type(
                lax.bitwise_and(w, jnp.int32(-65536)), jnp.float32)
            accs[d] = accs[d] + a
            accs[d + 4] = accs[d + 4] + b
          accs = tuple(accs)
        return accs
      return lax.fori_loop(0, HALF // 4, body, accs)

    for slot in range(5):
      gather_half(slot, slot)

    def outer(g, carry):
      r0 = 6 * g
      accs = None
      for s in range(6):
        @pl.when(r0 + s + 5 < ROWS_PER_W)
        def _(s=s):
          gather_half(r0 + s + 5, (s + 5) % 6)

        wait_half(s)
        if s % 2 == 0:
          accs = tuple(jnp.zeros((16,), jnp.float32) for _ in range(8))
        accs = reduce_half(bufs[s], accs)
        if s % 2 == 1:
          b = 3 * g + s // 2
          for d in range(8):
            m_local[b, d * 16:(d + 1) * 16] = accs[d]
      return carry

    n_full = ROWS_PER_W // 6
    lax.fori_loop(0, n_full, outer, 0)
    accs = None
    for s in range(ROWS_PER_W % 6):
      wait_half(s)
      if s % 2 == 0:
        accs = tuple(jnp.zeros((16,), jnp.float32) for _ in range(8))
      accs = reduce_half(bufs[s], accs)
      if s % 2 == 1:
        b = 3 * n_full + s // 2
        for d in range(8):
          m_local[b, d * 16:(d + 1) * 16] = accs[d]
    pltpu.sync_copy(m_local, out_hbm.at[pl.ds(wid * B_PER_W, B_PER_W)])

  return k(embed, xr)


def _mlp_kernel(msum_ref, w1_ref, b1_ref, g_ref, be_ref, w2_ref, b2_ref,
                out_ref):
  m = msum_ref[...] * (1.0 / SEQ)
  h = lax.dot_general(m, w1_ref[...], (((1,), (1,)), ((), ())),
                      preferred_element_type=jnp.float32)
  h = h + b1_ref[...]
  mu = jnp.mean(h, axis=0, keepdims=True)
  d = h - mu
  var = jnp.mean(d * d, axis=0, keepdims=True)
  hn = d * lax.rsqrt(var + 1e-5) * g_ref[...] + be_ref[...]
  a = jnp.maximum(hn, 0.0)
  out_ref[...] = lax.dot_general(a, w2_ref[...], (((1,), (1,)), ((), ())),
                                 preferred_element_type=jnp.float32) + b2_ref[...]


def kernel(X, embed, W1, b1, gamma, beta, W2, b2):
  xi = X.astype(jnp.int32)
  xr = jnp.where(xi < VOCAB // 2, 2 * xi,
                 2 * xi - (VOCAB - 1)).reshape(BATCH * 2, HALF)
  packed = _pack_table(embed).reshape(VOCAB, VEC_DIM // 2)
  msum = _sc_gather_sum(packed, xr)
  out = pl.pallas_call(
      _mlp_kernel,
      out_shape=jax.ShapeDtypeStruct((BATCH, LABELS), jnp.float32),
  )(msum, W1, b1.reshape(1, HIDDEN), gamma.reshape(1, HIDDEN),
    beta.reshape(1, HIDDEN), W2, b2.reshape(1, LABELS))
  return out
